# baseline (speedup 1.0000x reference)
import functools

import jax
import jax.numpy as jnp
from jax import lax
from jax.experimental import pallas as pl
from jax.experimental.pallas import tpu as pltpu
from jax.experimental.pallas import tpu_sc as plsc

N = 10000
NPAD = 10240
HALF = 5120
ACCR = 5248
E = 320000
DF = 128
DH = 64
NCLS = 47
W = 128
K = 10
NCORES = 2
NSUB = 16
B = 128
NBLK = 160
EPT = NBLK * B
EPAD = NSUB * EPT
RPT = ACCR // NSUB
TCB = 1280


def _sc_mesh():
    return plsc.VectorSubcoreMesh(core_axis_name="c", subcore_axis_name="s")


def _spmm_body(v_hbm, row_hbm, col_hbm, out_hbm,
               acc, idx_row, idx_col, msg0, msg1,
               gs0, gs1, ss0, ss1):
    c = lax.axis_index("c")
    s = lax.axis_index("s")
    msgs = (msg0, msg1)
    gsems = (gs0, gs1)
    ssems = (ss0, ss1)
    pltpu.sync_copy(row_hbm.at[s], idx_row.at[pl.ds(0, NBLK)])
    pltpu.sync_copy(col_hbm.at[s], idx_col)

    @pl.loop(NBLK, NBLK + 2)
    def _(r):
        @pl.loop(0, B, step=16)
        def _(i):
            idx_row[r, pl.ds(i, 16)] = jnp.zeros((16,), jnp.int32)

    basec = (c * HALF).astype(jnp.int32)
    dump0 = jnp.int32(HALF) + (s * 8).astype(jnp.int32)

    @pl.loop(0, NBLK)
    def _(j):
        @pl.loop(0, B, step=16)
        def _(i):
            d = idx_col[j, pl.ds(i, 16)] - basec
            ok = jnp.logical_and(d >= 0, d < HALF)
            dump = dump0 + lax.rem(lax.div(i, 16), jnp.int32(8))
            idx_col[j, pl.ds(i, 16)] = jnp.where(ok, d, dump)

    @pl.loop(0, B)
    def _(i):
        @pl.loop(0, W, step=16)
        def _(j):
            msg0[i, pl.ds(j, 16)] = jnp.zeros((16,), jnp.float32)

    base = s * RPT
    pltpu.sync_copy(msg0, acc.at[pl.ds(base, B)])
    pltpu.sync_copy(msg0, acc.at[pl.ds(base + B, B)])
    pltpu.sync_copy(msg0.at[pl.ds(0, RPT - 2 * B)],
                    acc.at[pl.ds(base + 2 * B, RPT - 2 * B)])

    for b in range(2):
        pltpu.async_copy(v_hbm.at[idx_row.at[b]], msgs[b], gsems[b])
    plsc.subcore_barrier()

    @pl.loop(0, NBLK, step=2)
    def _(jj):
        for b in range(2):
            pltpu.make_async_copy(
                v_hbm.at[idx_row.at[jj + b]], msgs[b], gsems[b]).wait()
            pltpu.async_copy(msgs[b], acc.at[idx_col.at[jj + b]],
                             ssems[b], add=True)
        for b in range(2):
            pltpu.make_async_copy(
                msgs[b], acc.at[idx_col.at[jj + b]], ssems[b]).wait()
            pltpu.async_copy(v_hbm.at[idx_row.at[jj + 2 + b]],
                             msgs[b], gsems[b])

    for b in range(2):
        pltpu.make_async_copy(
            v_hbm.at[idx_row.at[NBLK + b]], msgs[b], gsems[b]).wait()

    plsc.subcore_barrier()
    pltpu.sync_copy(acc.at[pl.ds(base, RPT)],
                    out_hbm.at[c, pl.ds(base, RPT)])


def _spmm(v, rowp, colp):
    k = pl.kernel(
        _spmm_body,
        out_type=jax.ShapeDtypeStruct((NCORES, ACCR, W), jnp.float32),
        mesh=_sc_mesh(),
        scratch_types=[
            pltpu.VMEM_SHARED((ACCR, W), jnp.float32),
            pltpu.VMEM((NBLK + 2, B), jnp.int32),
            pltpu.VMEM((NBLK, B), jnp.int32),
            pltpu.VMEM((B, W), jnp.float32),
            pltpu.VMEM((B, W), jnp.float32),
            pltpu.SemaphoreType.DMA,
            pltpu.SemaphoreType.DMA,
            pltpu.SemaphoreType.DMA,
            pltpu.SemaphoreType.DMA,
        ],
    )
    return k(v, rowp, colp)


def _mlp_body(x_ref, w1_ref, w2_ref, p_ref, o_ref):
    h = jnp.dot(x_ref[...], w1_ref[...], preferred_element_type=jnp.float32)
    h = jnp.maximum(h + p_ref[0:1, 0:DH], 0.0)
    o_ref[...] = (jnp.dot(h, w2_ref[...], preferred_element_type=jnp.float32)
                  + p_ref[1:2, :])


def _mlp(xp, W1, W2p, params):
    return pl.pallas_call(
        _mlp_body,
        grid=(NPAD // TCB,),
        in_specs=[
            pl.BlockSpec((TCB, DF), lambda i: (i, 0)),
            pl.BlockSpec((DF, DH), lambda i: (0, 0)),
            pl.BlockSpec((DH, W), lambda i: (0, 0)),
            pl.BlockSpec((8, 128), lambda i: (0, 0)),
        ],
        out_specs=pl.BlockSpec((TCB, W), lambda i: (i, 0)),
        out_shape=jax.ShapeDtypeStruct((NPAD, W), jnp.float32),
    )(xp, W1, W2p, params)


def _a_spec():
    return pl.BlockSpec((1, TCB, W), lambda i: (i // 4, i % 4, 0))


def _combine_body(degp_ref, h0_ref, p_ref, dinv_ref, v0_ref, hid_ref):
    deg = degp_ref[0, :, 0:1] + 1.0
    dinv = lax.rsqrt(deg)
    dinv_ref[...] = dinv
    h0 = h0_ref[...]
    v0_ref[...] = dinv * h0
    hid_ref[...] = p_ref[2:3, 0:1] * h0


def _combine(degp, h0, params):
    return pl.pallas_call(
        _combine_body,
        grid=(NPAD // TCB,),
        in_specs=[
            _a_spec(),
            pl.BlockSpec((TCB, W), lambda i: (i, 0)),
            pl.BlockSpec((8, 128), lambda i: (0, 0)),
        ],
        out_specs=[
            pl.BlockSpec((TCB, 1), lambda i: (i, 0)),
            pl.BlockSpec((TCB, W), lambda i: (i, 0)),
            pl.BlockSpec((TCB, W), lambda i: (i, 0)),
        ],
        out_shape=[
            jax.ShapeDtypeStruct((NPAD, 1), jnp.float32),
            jax.ShapeDtypeStruct((NPAD, W), jnp.float32),
            jax.ShapeDtypeStruct((NPAD, W), jnp.float32),
        ],
    )(degp, h0, params)


def _update_body(k, a_ref, v_ref, dinv_ref, hid_ref, p_ref, v_out, hid_out):
    sm = a_ref[0] + v_ref[...]
    dinv = dinv_ref[...]
    t = dinv * sm
    hid_out[...] = hid_ref[...] + p_ref[2:3, k + 1:k + 2] * t
    v_out[...] = dinv * t


def _update(k, a, v, dinv, hid, params):
    return pl.pallas_call(
        functools.partial(_update_body, k),
        grid=(NPAD // TCB,),
        in_specs=[
            _a_spec(),
            pl.BlockSpec((TCB, W), lambda i: (i, 0)),
            pl.BlockSpec((TCB, 1), lambda i: (i, 0)),
            pl.BlockSpec((TCB, W), lambda i: (i, 0)),
            pl.BlockSpec((8, 128), lambda i: (0, 0)),
        ],
        out_specs=[
            pl.BlockSpec((TCB, W), lambda i: (i, 0)),
            pl.BlockSpec((TCB, W), lambda i: (i, 0)),
        ],
        out_shape=[
            jax.ShapeDtypeStruct((NPAD, W), jnp.float32),
            jax.ShapeDtypeStruct((NPAD, W), jnp.float32),
        ],
    )(a, v, dinv, hid, params)


def _lsm_body(hid_ref, o_ref):
    x = hid_ref[...]
    colid = lax.broadcasted_iota(jnp.int32, (1000, W), 1)
    xm = jnp.where(colid < NCLS, x, -1e30)
    m = jnp.max(xm, axis=1, keepdims=True)
    e = jnp.exp(xm - m)
    lse = jnp.log(jnp.sum(e, axis=1, keepdims=True)) + m
    o_ref[...] = x[:, :NCLS] - lse


def _lsm(hid):
    return pl.pallas_call(
        _lsm_body,
        grid=(N // 1000,),
        in_specs=[pl.BlockSpec((1000, W), lambda i: (i, 0))],
        out_specs=pl.BlockSpec((1000, NCLS), lambda i: (i, 0)),
        out_shape=jax.ShapeDtypeStruct((N, NCLS), jnp.float32),
    )(hid)


def kernel(x, edge_index, W1, b1, W2, b2, temp):
    gamma = jax.nn.relu(temp)
    row = edge_index[0]
    col = edge_index[1]
    rowp = jnp.concatenate(
        [row, jnp.zeros((EPAD - E,), jnp.int32)]).reshape(NSUB, NBLK, B)
    colp = jnp.concatenate(
        [col, jnp.full((EPAD - E,), NPAD - 1, jnp.int32)]).reshape(NSUB, NBLK, B)

    xp = jnp.pad(x, ((0, NPAD - N), (0, 0)))
    W2p = jnp.pad(W2, ((0, 0), (0, W - NCLS)))
    params = jnp.zeros((8, 128), jnp.float32)
    params = params.at[0, :DH].set(b1)
    params = params.at[1, :NCLS].set(b2)
    params = params.at[2, :K + 1].set(gamma)

    ones = jnp.ones((NPAD, W), jnp.float32)
    h0 = _mlp(xp, W1, W2p, params)
    degp = _spmm(ones, rowp, colp)
    dinv, v, hid = _combine(degp, h0, params)
    for k in range(K):
        a = _spmm(v, rowp, colp)
        v, hid = _update(k, a, v, dinv, hid, params)
    return _lsm(hid)


# speedup vs baseline: 1.9650x; 1.9650x over previous
import dataclasses
import functools

import jax
import jax.numpy as jnp
from jax import lax
from jax.experimental import pallas as pl
from jax.experimental.pallas import tpu as pltpu
from jax.experimental.pallas import tpu_sc as plsc

N = 10000
NPAD = 10240
HALF = 5120
ACCR = 5248
E = 320000
DF = 128
DH = 64
NCLS = 47
W = 128
K = 10
NCORES = 2
NSUB = 16
B = 128
NBLK = 160
CHBLK = 80
CAPB = NBLK + 2
CAPE = CAPB * B
EPT = CHBLK * B
EPAD = NCORES * NSUB * EPT
RPT = ACCR // NSUB
TCB = 1280


def _sc_mesh():
    return plsc.VectorSubcoreMesh(core_axis_name="c", subcore_axis_name="s")


def _part_body(row_hbm, col_hbm, rowq_hbm, colq_hbm, cnt_hbm,
               rbuf, cbuf, srow, scol, scol2, crow):
    c = lax.axis_index("c")
    s = lax.axis_index("s")
    pltpu.sync_copy(row_hbm.at[2 * s], rbuf.at[pl.ds(0, CHBLK)])
    pltpu.sync_copy(row_hbm.at[2 * s + 1], rbuf.at[pl.ds(CHBLK, CHBLK)])
    pltpu.sync_copy(col_hbm.at[2 * s], cbuf.at[pl.ds(0, CHBLK)])
    pltpu.sync_copy(col_hbm.at[2 * s + 1], cbuf.at[pl.ds(CHBLK, CHBLK)])

    dumpvec = (jnp.int32(HALF) + (s * 8).astype(jnp.int32)
               + lax.rem(lax.iota(jnp.int32, 16), jnp.int32(8)))
    zerov = jnp.zeros((16,), jnp.int32)

    @pl.loop(0, CAPE + 16, step=16)
    def _(i):
        srow[pl.ds(i, 16)] = zerov
        scol[pl.ds(i, 16)] = dumpvec

    basec = (c * HALF).astype(jnp.int32)

    def body(g, cur):
        r = g // 8
        o = (g % 8) * 16
        d = cbuf[r, pl.ds(o, 16)] - basec
        ok = jnp.logical_and(d >= 0, d < HALF)
        plsc.store_compressed(scol.at[pl.ds(cur, 16)], d, mask=ok)
        plsc.store_compressed(srow.at[pl.ds(cur, 16)], rbuf[r, pl.ds(o, 16)], mask=ok)
        return cur + jnp.max(plsc.all_reduce_population_count(ok))

    total = lax.fori_loop(0, 2 * CHBLK * 8, body, jnp.int32(0))
    nblk = (total + jnp.int32(2 * B - 1)) // jnp.int32(2 * B) * 2

    @pl.loop(0, 128, step=16)
    def _(i):
        crow[0, pl.ds(i, 16)] = jnp.broadcast_to(nblk, (16,))

    @pl.loop(0, CAPB)
    def _(j):
        @pl.loop(0, B, step=16)
        def _(i):
            scol2[j, pl.ds(i, 16)] = scol[pl.ds(j * B + i, 16)]

    pltpu.sync_copy(crow, cnt_hbm.at[c, s])
    pltpu.sync_copy(srow.at[pl.ds(0, CAPE)], rowq_hbm.at[c, s])
    pltpu.sync_copy(scol2, colq_hbm.at[c, s])


def _part(rowp, colp):
    cp = dataclasses.replace(pltpu.CompilerParams(), needs_layout_passes=False)
    k = pl.kernel(
        _part_body,
        compiler_params=cp,
        out_type=[
            jax.ShapeDtypeStruct((NCORES, NSUB, CAPE), jnp.int32),
            jax.ShapeDtypeStruct((NCORES, NSUB, CAPB, B), jnp.int32),
            jax.ShapeDtypeStruct((NCORES, NSUB, 1, 128), jnp.int32),
        ],
        mesh=_sc_mesh(),
        scratch_types=[
            pltpu.VMEM((2 * CHBLK, B), jnp.int32),
            pltpu.VMEM((2 * CHBLK, B), jnp.int32),
            pltpu.VMEM((CAPE + 16,), jnp.int32),
            pltpu.VMEM((CAPE + 16,), jnp.int32),
            pltpu.VMEM((CAPB, B), jnp.int32),
            pltpu.VMEM((1, 128), jnp.int32),
        ],
    )
    return k(rowp, colp)


def _spmm_body(v_hbm, rowq_hbm, colq_hbm, cnt_hbm, out_hbm,
               acc, idx_row, idx_col, msg0, msg1, crow, gs0, gs1):
    c = lax.axis_index("c")
    s = lax.axis_index("s")
    msgs = (msg0, msg1)
    gsems = (gs0, gs1)
    pltpu.sync_copy(rowq_hbm.at[c, s], idx_row)
    pltpu.sync_copy(colq_hbm.at[c, s], idx_col)
    pltpu.sync_copy(cnt_hbm.at[c, s], crow)
    nblk = crow[0, pl.ds(0, 16)][0]

    @pl.loop(0, B)
    def _(i):
        @pl.loop(0, W, step=16)
        def _(j):
            msg0[i, pl.ds(j, 16)] = jnp.zeros((16,), jnp.float32)

    base = s * RPT
    pltpu.sync_copy(msg0, acc.at[pl.ds(base, B)])
    pltpu.sync_copy(msg0, acc.at[pl.ds(base + B, B)])
    pltpu.sync_copy(msg0.at[pl.ds(0, RPT - 2 * B)],
                    acc.at[pl.ds(base + 2 * B, RPT - 2 * B)])

    for b in range(2):
        pltpu.async_copy(v_hbm.at[idx_row.at[pl.ds(b * B, B)]],
                         msgs[b], gsems[b])
    plsc.subcore_barrier()

    @pl.loop(0, nblk, step=2)
    def _(jj):
        for b in range(2):
            pltpu.make_async_copy(
                v_hbm.at[idx_row.at[pl.ds((jj + b) * B, B)]],
                msgs[b], gsems[b]).wait()
            pltpu.sync_copy(msgs[b], acc.at[idx_col.at[jj + b]], add=True)
            pltpu.async_copy(v_hbm.at[idx_row.at[pl.ds((jj + 2 + b) * B, B)]],
                             msgs[b], gsems[b])

    for b in range(2):
        pltpu.make_async_copy(v_hbm.at[idx_row.at[pl.ds(b * B, B)]],
                              msgs[b], gsems[b]).wait()

    plsc.subcore_barrier()
    pltpu.sync_copy(acc.at[pl.ds(base, RPT)],
                    out_hbm.at[c, pl.ds(base, RPT)])


def _spmm(v, rowq, colq, cnts):
    k = pl.kernel(
        _spmm_body,
        out_type=jax.ShapeDtypeStruct((NCORES, ACCR, W), jnp.float32),
        mesh=_sc_mesh(),
        scratch_types=[
            pltpu.VMEM_SHARED((ACCR, W), jnp.float32),
            pltpu.VMEM((CAPE,), jnp.int32),
            pltpu.VMEM((CAPB, B), jnp.int32),
            pltpu.VMEM((B, W), jnp.float32),
            pltpu.VMEM((B, W), jnp.float32),
            pltpu.VMEM((1, 128), jnp.int32),
            pltpu.SemaphoreType.DMA,
            pltpu.SemaphoreType.DMA,
        ],
    )
    return k(v, rowq, colq, cnts)


def _mlp_body(x_ref, w1_ref, w2_ref, p_ref, o_ref):
    h = jnp.dot(x_ref[...], w1_ref[...], preferred_element_type=jnp.float32)
    h = jnp.maximum(h + p_ref[0:1, 0:DH], 0.0)
    o_ref[...] = (jnp.dot(h, w2_ref[...], preferred_element_type=jnp.float32)
                  + p_ref[1:2, :])


def _mlp(xp, W1, W2p, params):
    return pl.pallas_call(
        _mlp_body,
        grid=(NPAD // TCB,),
        in_specs=[
            pl.BlockSpec((TCB, DF), lambda i: (i, 0)),
            pl.BlockSpec((DF, DH), lambda i: (0, 0)),
            pl.BlockSpec((DH, W), lambda i: (0, 0)),
            pl.BlockSpec((8, 128), lambda i: (0, 0)),
        ],
        out_specs=pl.BlockSpec((TCB, W), lambda i: (i, 0)),
        out_shape=jax.ShapeDtypeStruct((NPAD, W), jnp.float32),
    )(xp, W1, W2p, params)


def _a_spec():
    return pl.BlockSpec((1, TCB, W), lambda i: (i // 4, i % 4, 0))


def _combine_body(degp_ref, h0_ref, p_ref, dinv_ref, v0_ref, hid_ref):
    deg = degp_ref[0, :, 0:1] + 1.0
    dinv = lax.rsqrt(deg)
    dinv_ref[...] = dinv
    h0 = h0_ref[...]
    v0_ref[...] = dinv * h0
    hid_ref[...] = p_ref[2:3, 0:1] * h0


def _combine(degp, h0, params):
    return pl.pallas_call(
        _combine_body,
        grid=(NPAD // TCB,),
        in_specs=[
            _a_spec(),
            pl.BlockSpec((TCB, W), lambda i: (i, 0)),
            pl.BlockSpec((8, 128), lambda i: (0, 0)),
        ],
        out_specs=[
            pl.BlockSpec((TCB, 1), lambda i: (i, 0)),
            pl.BlockSpec((TCB, W), lambda i: (i, 0)),
            pl.BlockSpec((TCB, W), lambda i: (i, 0)),
        ],
        out_shape=[
            jax.ShapeDtypeStruct((NPAD, 1), jnp.float32),
            jax.ShapeDtypeStruct((NPAD, W), jnp.float32),
            jax.ShapeDtypeStruct((NPAD, W), jnp.float32),
        ],
    )(degp, h0, params)


def _update_body(k, a_ref, v_ref, dinv_ref, hid_ref, p_ref, v_out, hid_out):
    sm = a_ref[0] + v_ref[...]
    dinv = dinv_ref[...]
    t = dinv * sm
    hid_out[...] = hid_ref[...] + p_ref[2:3, k + 1:k + 2] * t
    v_out[...] = dinv * t


def _update(k, a, v, dinv, hid, params):
    return pl.pallas_call(
        functools.partial(_update_body, k),
        grid=(NPAD // TCB,),
        in_specs=[
            _a_spec(),
            pl.BlockSpec((TCB, W), lambda i: (i, 0)),
            pl.BlockSpec((TCB, 1), lambda i: (i, 0)),
            pl.BlockSpec((TCB, W), lambda i: (i, 0)),
            pl.BlockSpec((8, 128), lambda i: (0, 0)),
        ],
        out_specs=[
            pl.BlockSpec((TCB, W), lambda i: (i, 0)),
            pl.BlockSpec((TCB, W), lambda i: (i, 0)),
        ],
        out_shape=[
            jax.ShapeDtypeStruct((NPAD, W), jnp.float32),
            jax.ShapeDtypeStruct((NPAD, W), jnp.float32),
        ],
    )(a, v, dinv, hid, params)


def _lsm_body(hid_ref, o_ref):
    x = hid_ref[...]
    colid = lax.broadcasted_iota(jnp.int32, (1000, W), 1)
    xm = jnp.where(colid < NCLS, x, -1e30)
    m = jnp.max(xm, axis=1, keepdims=True)
    e = jnp.exp(xm - m)
    lse = jnp.log(jnp.sum(e, axis=1, keepdims=True)) + m
    o_ref[...] = x[:, :NCLS] - lse


def _lsm(hid):
    return pl.pallas_call(
        _lsm_body,
        grid=(N // 1000,),
        in_specs=[pl.BlockSpec((1000, W), lambda i: (i, 0))],
        out_specs=pl.BlockSpec((1000, NCLS), lambda i: (i, 0)),
        out_shape=jax.ShapeDtypeStruct((N, NCLS), jnp.float32),
    )(hid)


def kernel(x, edge_index, W1, b1, W2, b2, temp):
    gamma = jax.nn.relu(temp)
    row = edge_index[0]
    col = edge_index[1]
    rowp = jnp.concatenate(
        [row, jnp.zeros((EPAD - E,), jnp.int32)]).reshape(NCORES * NSUB, CHBLK, B)
    colp = jnp.concatenate(
        [col, jnp.full((EPAD - E,), NPAD, jnp.int32)]).reshape(NCORES * NSUB, CHBLK, B)

    xp = jnp.pad(x, ((0, NPAD - N), (0, 0)))
    W2p = jnp.pad(W2, ((0, 0), (0, W - NCLS)))
    params = jnp.zeros((8, 128), jnp.float32)
    params = params.at[0, :DH].set(b1)
    params = params.at[1, :NCLS].set(b2)
    params = params.at[2, :K + 1].set(gamma)

    ones = jnp.ones((NPAD, W), jnp.float32)
    h0 = _mlp(xp, W1, W2p, params)
    rowq, colq, cnts = _part(rowp, colp)
    degp = _spmm(ones, rowq, colq, cnts)
    dinv, v, hid = _combine(degp, h0, params)
    for k in range(K):
        a = _spmm(v, rowq, colq, cnts)
        v, hid = _update(k, a, v, dinv, hid, params)
    return _lsm(hid)


# speedup vs baseline: 2.1322x; 1.0851x over previous
import dataclasses
import functools

import jax
import jax.numpy as jnp
from jax import lax
from jax.experimental import pallas as pl
from jax.experimental.pallas import tpu as pltpu
from jax.experimental.pallas import tpu_sc as plsc

N = 10000
NPAD = 10240
HALF = 5120
ACCR = 5248
E = 320000
DF = 128
DH = 64
NCLS = 47
W = 128
K = 10
NCORES = 2
NSUB = 16
B = 128
NBLK = 160
CHBLK = 80
CAPB = NBLK + 2
CAPE = CAPB * B
EPT = CHBLK * B
EPAD = NCORES * NSUB * EPT
RPT = ACCR // NSUB
TCB = 1280


def _sc_mesh():
    return plsc.VectorSubcoreMesh(core_axis_name="c", subcore_axis_name="s")


def _part_body(row_hbm, col_hbm, rowq_hbm, colq_hbm, cnt_hbm,
               rbuf, cbuf, srow, scol, scol2, crow):
    c = lax.axis_index("c")
    s = lax.axis_index("s")
    pltpu.sync_copy(row_hbm.at[2 * s], rbuf.at[pl.ds(0, CHBLK)])
    pltpu.sync_copy(row_hbm.at[2 * s + 1], rbuf.at[pl.ds(CHBLK, CHBLK)])
    pltpu.sync_copy(col_hbm.at[2 * s], cbuf.at[pl.ds(0, CHBLK)])
    pltpu.sync_copy(col_hbm.at[2 * s + 1], cbuf.at[pl.ds(CHBLK, CHBLK)])

    dumpvec = (jnp.int32(HALF) + (s * 8).astype(jnp.int32)
               + lax.rem(lax.iota(jnp.int32, 16), jnp.int32(8)))
    zerov = jnp.zeros((16,), jnp.int32)

    @pl.loop(0, CAPE + 16, step=16)
    def _(i):
        srow[pl.ds(i, 16)] = zerov
        scol[pl.ds(i, 16)] = dumpvec

    basec = (c * HALF).astype(jnp.int32)

    def body(g, cur):
        r = g // 8
        o = (g % 8) * 16
        d = cbuf[r, pl.ds(o, 16)] - basec
        ok = jnp.logical_and(d >= 0, d < HALF)
        plsc.store_compressed(scol.at[pl.ds(cur, 16)], d, mask=ok)
        plsc.store_compressed(srow.at[pl.ds(cur, 16)], rbuf[r, pl.ds(o, 16)], mask=ok)
        return cur + jnp.max(plsc.all_reduce_population_count(ok))

    total = lax.fori_loop(0, 2 * CHBLK * 8, body, jnp.int32(0))
    nblk = (total + jnp.int32(2 * B - 1)) // jnp.int32(2 * B) * 2

    @pl.loop(0, 128, step=16)
    def _(i):
        crow[0, pl.ds(i, 16)] = jnp.broadcast_to(nblk, (16,))

    @pl.loop(0, CAPB)
    def _(j):
        @pl.loop(0, B, step=16)
        def _(i):
            scol2[j, pl.ds(i, 16)] = scol[pl.ds(j * B + i, 16)]

    pltpu.sync_copy(crow, cnt_hbm.at[c, s])
    pltpu.sync_copy(srow.at[pl.ds(0, CAPE)], rowq_hbm.at[c, s])
    pltpu.sync_copy(scol2, colq_hbm.at[c, s])


def _part(rowp, colp):
    cp = dataclasses.replace(pltpu.CompilerParams(), needs_layout_passes=False)
    k = pl.kernel(
        _part_body,
        compiler_params=cp,
        out_type=[
            jax.ShapeDtypeStruct((NCORES, NSUB, CAPE), jnp.int32),
            jax.ShapeDtypeStruct((NCORES, NSUB, CAPB, B), jnp.int32),
            jax.ShapeDtypeStruct((NCORES, NSUB, 1, 128), jnp.int32),
        ],
        mesh=_sc_mesh(),
        scratch_types=[
            pltpu.VMEM((2 * CHBLK, B), jnp.int32),
            pltpu.VMEM((2 * CHBLK, B), jnp.int32),
            pltpu.VMEM((CAPE + 16,), jnp.int32),
            pltpu.VMEM((CAPE + 16,), jnp.int32),
            pltpu.VMEM((CAPB, B), jnp.int32),
            pltpu.VMEM((1, 128), jnp.int32),
        ],
    )
    return k(rowp, colp)


def _spmm_body(v_hbm, rowq_hbm, colq_hbm, cnt_hbm, out_hbm,
               acc, idx_row, idx_col, msg0, msg1, crow, gs0, gs1):
    c = lax.axis_index("c")
    s = lax.axis_index("s")
    msgs = (msg0, msg1)
    gsems = (gs0, gs1)
    pltpu.sync_copy(rowq_hbm.at[c, s], idx_row)
    pltpu.sync_copy(colq_hbm.at[c, s], idx_col)
    pltpu.sync_copy(cnt_hbm.at[c, s], crow)
    nblk = crow[0, pl.ds(0, 16)][0]

    @pl.loop(0, B)
    def _(i):
        @pl.loop(0, W, step=16)
        def _(j):
            msg0[i, pl.ds(j, 16)] = jnp.zeros((16,), jnp.float32)

    base = s * RPT
    pltpu.sync_copy(msg0, acc.at[pl.ds(base, B)])
    pltpu.sync_copy(msg0, acc.at[pl.ds(base + B, B)])
    pltpu.sync_copy(msg0.at[pl.ds(0, RPT - 2 * B)],
                    acc.at[pl.ds(base + 2 * B, RPT - 2 * B)])

    for b in range(2):
        pltpu.async_copy(v_hbm.at[idx_row.at[pl.ds(b * B, B)]],
                         msgs[b], gsems[b])
    plsc.subcore_barrier()

    @pl.loop(0, nblk, step=2)
    def _(jj):
        for b in range(2):
            pltpu.make_async_copy(
                v_hbm.at[idx_row.at[pl.ds((jj + b) * B, B)]],
                msgs[b], gsems[b]).wait()
            pltpu.sync_copy(msgs[b], acc.at[idx_col.at[jj + b]], add=True)
            pltpu.async_copy(v_hbm.at[idx_row.at[pl.ds((jj + 2 + b) * B, B)]],
                             msgs[b], gsems[b])

    for b in range(2):
        pltpu.make_async_copy(v_hbm.at[idx_row.at[pl.ds(b * B, B)]],
                              msgs[b], gsems[b]).wait()

    plsc.subcore_barrier()
    pltpu.sync_copy(acc.at[pl.ds(base, RPT)],
                    out_hbm.at[c, pl.ds(base, RPT)])


def _spmm(v, rowq, colq, cnts):
    k = pl.kernel(
        _spmm_body,
        out_type=jax.ShapeDtypeStruct((NCORES, ACCR, W), jnp.float32),
        mesh=_sc_mesh(),
        scratch_types=[
            pltpu.VMEM_SHARED((ACCR, W), jnp.float32),
            pltpu.VMEM((CAPE,), jnp.int32),
            pltpu.VMEM((CAPB, B), jnp.int32),
            pltpu.VMEM((B, W), jnp.float32),
            pltpu.VMEM((B, W), jnp.float32),
            pltpu.VMEM((1, 128), jnp.int32),
            pltpu.SemaphoreType.DMA,
            pltpu.SemaphoreType.DMA,
        ],
    )
    return k(v, rowq, colq, cnts)


def _deg_body(colq_hbm, cnt_hbm, out_hbm, acc, idx_col, msg0, crow):
    c = lax.axis_index("c")
    s = lax.axis_index("s")
    pltpu.sync_copy(colq_hbm.at[c, s], idx_col)
    pltpu.sync_copy(cnt_hbm.at[c, s], crow)
    nblk = crow[0, pl.ds(0, 16)][0]

    @pl.loop(0, B)
    def _(i):
        @pl.loop(0, W, step=16)
        def _(j):
            msg0[i, pl.ds(j, 16)] = jnp.zeros((16,), jnp.float32)

    base = s * RPT
    pltpu.sync_copy(msg0, acc.at[pl.ds(base, B)])
    pltpu.sync_copy(msg0, acc.at[pl.ds(base + B, B)])
    pltpu.sync_copy(msg0.at[pl.ds(0, RPT - 2 * B)],
                    acc.at[pl.ds(base + 2 * B, RPT - 2 * B)])

    @pl.loop(0, B)
    def _(i):
        @pl.loop(0, W, step=16)
        def _(j):
            msg0[i, pl.ds(j, 16)] = jnp.ones((16,), jnp.float32)

    plsc.subcore_barrier()

    @pl.loop(0, nblk)
    def _(j):
        pltpu.sync_copy(msg0, acc.at[idx_col.at[j]], add=True)

    plsc.subcore_barrier()
    pltpu.sync_copy(acc.at[pl.ds(base, RPT)],
                    out_hbm.at[c, pl.ds(base, RPT)])


def _deg(colq, cnts):
    k = pl.kernel(
        _deg_body,
        out_type=jax.ShapeDtypeStruct((NCORES, ACCR, W), jnp.float32),
        mesh=_sc_mesh(),
        scratch_types=[
            pltpu.VMEM_SHARED((ACCR, W), jnp.float32),
            pltpu.VMEM((CAPB, B), jnp.int32),
            pltpu.VMEM((B, W), jnp.float32),
            pltpu.VMEM((1, 128), jnp.int32),
        ],
    )
    return k(colq, cnts)


def _mlp_body(x_ref, w1_ref, w2_ref, p_ref, o_ref):
    h = jnp.dot(x_ref[...], w1_ref[...], preferred_element_type=jnp.float32)
    h = jnp.maximum(h + p_ref[0:1, 0:DH], 0.0)
    o_ref[...] = (jnp.dot(h, w2_ref[...], preferred_element_type=jnp.float32)
                  + p_ref[1:2, :])


def _mlp(xp, W1, W2p, params):
    return pl.pallas_call(
        _mlp_body,
        grid=(NPAD // TCB,),
        in_specs=[
            pl.BlockSpec((TCB, DF), lambda i: (i, 0)),
            pl.BlockSpec((DF, DH), lambda i: (0, 0)),
            pl.BlockSpec((DH, W), lambda i: (0, 0)),
            pl.BlockSpec((8, 128), lambda i: (0, 0)),
        ],
        out_specs=pl.BlockSpec((TCB, W), lambda i: (i, 0)),
        out_shape=jax.ShapeDtypeStruct((NPAD, W), jnp.float32),
    )(xp, W1, W2p, params)


def _a_spec():
    return pl.BlockSpec((1, TCB, W), lambda i: (i // 4, i % 4, 0))


def _combine_body(degp_ref, h0_ref, p_ref, dinv_ref, v0_ref, hid_ref):
    deg = degp_ref[0, :, 0:1] + 1.0
    dinv = lax.rsqrt(deg)
    dinv_ref[...] = dinv
    h0 = h0_ref[...]
    v0_ref[...] = dinv * h0
    hid_ref[...] = p_ref[2:3, 0:1] * h0


def _combine(degp, h0, params):
    return pl.pallas_call(
        _combine_body,
        grid=(NPAD // TCB,),
        in_specs=[
            _a_spec(),
            pl.BlockSpec((TCB, W), lambda i: (i, 0)),
            pl.BlockSpec((8, 128), lambda i: (0, 0)),
        ],
        out_specs=[
            pl.BlockSpec((TCB, 1), lambda i: (i, 0)),
            pl.BlockSpec((TCB, W), lambda i: (i, 0)),
            pl.BlockSpec((TCB, W), lambda i: (i, 0)),
        ],
        out_shape=[
            jax.ShapeDtypeStruct((NPAD, 1), jnp.float32),
            jax.ShapeDtypeStruct((NPAD, W), jnp.float32),
            jax.ShapeDtypeStruct((NPAD, W), jnp.float32),
        ],
    )(degp, h0, params)


def _update_body(k, a_ref, v_ref, dinv_ref, hid_ref, p_ref, v_out, hid_out):
    sm = a_ref[0] + v_ref[...]
    dinv = dinv_ref[...]
    t = dinv * sm
    hid_out[...] = hid_ref[...] + p_ref[2:3, k + 1:k + 2] * t
    v_out[...] = dinv * t


def _update(k, a, v, dinv, hid, params):
    return pl.pallas_call(
        functools.partial(_update_body, k),
        grid=(NPAD // TCB,),
        in_specs=[
            _a_spec(),
            pl.BlockSpec((TCB, W), lambda i: (i, 0)),
            pl.BlockSpec((TCB, 1), lambda i: (i, 0)),
            pl.BlockSpec((TCB, W), lambda i: (i, 0)),
            pl.BlockSpec((8, 128), lambda i: (0, 0)),
        ],
        out_specs=[
            pl.BlockSpec((TCB, W), lambda i: (i, 0)),
            pl.BlockSpec((TCB, W), lambda i: (i, 0)),
        ],
        out_shape=[
            jax.ShapeDtypeStruct((NPAD, W), jnp.float32),
            jax.ShapeDtypeStruct((NPAD, W), jnp.float32),
        ],
    )(a, v, dinv, hid, params)


def _lsm_body(hid_ref, o_ref):
    x = hid_ref[...]
    colid = lax.broadcasted_iota(jnp.int32, (1000, W), 1)
    xm = jnp.where(colid < NCLS, x, -1e30)
    m = jnp.max(xm, axis=1, keepdims=True)
    e = jnp.exp(xm - m)
    lse = jnp.log(jnp.sum(e, axis=1, keepdims=True)) + m
    o_ref[...] = x[:, :NCLS] - lse


def _lsm(hid):
    return pl.pallas_call(
        _lsm_body,
        grid=(N // 1000,),
        in_specs=[pl.BlockSpec((1000, W), lambda i: (i, 0))],
        out_specs=pl.BlockSpec((1000, NCLS), lambda i: (i, 0)),
        out_shape=jax.ShapeDtypeStruct((N, NCLS), jnp.float32),
    )(hid)


def kernel(x, edge_index, W1, b1, W2, b2, temp):
    gamma = jax.nn.relu(temp)
    row = edge_index[0]
    col = edge_index[1]
    rowp = jnp.concatenate(
        [row, jnp.zeros((EPAD - E,), jnp.int32)]).reshape(NCORES * NSUB, CHBLK, B)
    colp = jnp.concatenate(
        [col, jnp.full((EPAD - E,), NPAD, jnp.int32)]).reshape(NCORES * NSUB, CHBLK, B)

    xp = jnp.pad(x, ((0, NPAD - N), (0, 0)))
    W2p = jnp.pad(W2, ((0, 0), (0, W - NCLS)))
    params = jnp.zeros((8, 128), jnp.float32)
    params = params.at[0, :DH].set(b1)
    params = params.at[1, :NCLS].set(b2)
    params = params.at[2, :K + 1].set(gamma)

    h0 = _mlp(xp, W1, W2p, params)
    rowq, colq, cnts = _part(rowp, colp)
    degp = _deg(colq, cnts)
    dinv, v, hid = _combine(degp, h0, params)
    for k in range(K):
        a = _spmm(v, rowq, colq, cnts)
        v, hid = _update(k, a, v, dinv, hid, params)
    return _lsm(hid)
